# Initial kernel scaffold; baseline (speedup 1.0000x reference)
import jax
import jax.numpy as jnp
from jax.experimental import pallas as pl


def kernel(x, gate_W, logit_bias, null_logit, W_gate, W_up, W_down, sg_W, su_W, sd_W):
    raise NotImplementedError("write your pallas kernel here")


# speedup vs baseline: 1.1815x; 1.1815x over previous
import jax
import jax.numpy as jnp
from jax.experimental import pallas as pl
from jax.experimental.pallas import tpu as pltpu

_E = 8
_D = 1024
_H = 1024
_RHO = 0.5
_N = 2048
_TT = 512


def _router_kernel(x_ref, gwt_ref, bias_ref, vnull_ref, combine_ref, aux_ref):
    x = x_ref[...]
    gwt = gwt_ref[...]
    l = jnp.dot(x, gwt, preferred_element_type=jnp.float32) + bias_ref[...]
    v = vnull_ref[0, 0]

    idx = jax.lax.broadcasted_iota(jnp.int32, (_N, _E), 1)
    l1 = jnp.max(l, axis=-1, keepdims=True)
    i1 = jnp.min(jnp.where(l == l1, idx, _E), axis=-1, keepdims=True)
    oh1 = idx == i1
    s1_real = l1 >= v
    lm = jnp.where(oh1, -jnp.inf, l)
    l2 = jnp.max(lm, axis=-1, keepdims=True)
    i2 = jnp.min(jnp.where(lm == l2, idx, _E), axis=-1, keepdims=True)
    oh2 = idx == i2
    s2_real = s1_real & (l2 >= v)

    m = jnp.maximum(l1, v)
    el = jnp.exp(l - m)
    ev = jnp.exp(v - m)
    z = jnp.sum(el, axis=-1, keepdims=True) + 8.0 * ev
    p = el / z
    w1 = jnp.where(s1_real, jnp.sum(jnp.where(oh1, p, 0.0), axis=-1, keepdims=True), 0.0)
    w2 = jnp.where(s2_real, jnp.sum(jnp.where(oh2, p, 0.0), axis=-1, keepdims=True), 0.0)
    wsum = jnp.maximum(w1 + w2, 1e-6)
    combine_ref[...] = (jnp.where(oh1, w1, 0.0) + jnp.where(oh2, w2, 0.0)) / wsum

    elr = jnp.exp(l - l1)
    pr = elr / jnp.sum(elr, axis=-1, keepdims=True)
    p_real = jnp.mean(pr, axis=0)
    sel = (oh1 & s1_real).astype(jnp.float32) + (oh2 & s2_real).astype(jnp.float32)
    counts = jnp.sum(sel, axis=0)
    total = jnp.maximum(jnp.sum(counts), 1e-6)
    l_bal = _E * jnp.sum((counts / total) * p_real)
    n_real = jnp.sum(s1_real.astype(jnp.float32)) + jnp.sum(s2_real.astype(jnp.float32))
    null_rate = (2.0 * _N - n_real) / (2.0 * _N)
    l_null = (null_rate - _RHO) ** 2
    lse = m + jnp.log(z)
    l_z = jnp.mean(lse * lse)
    aux = 0.02 * l_bal + 0.001 * l_z + 0.01 * l_null
    aux_ref[...] = jnp.reshape(aux, (1, 1))


def _ffn_kernel(cmb_ref, x_ref, wg_ref, wu_ref, wd_ref, out_ref):
    e = pl.program_id(1)
    xb = x_ref[...]
    g = jnp.dot(xb, wg_ref[0], preferred_element_type=jnp.float32)
    u = jnp.dot(xb, wu_ref[0], preferred_element_type=jnp.float32)
    h = (g * jax.nn.sigmoid(g) * u).astype(jnp.bfloat16)
    y = jnp.dot(h, wd_ref[0], preferred_element_type=jnp.float32)
    wy = cmb_ref[0, 0, :][:, None] * y

    @pl.when(e == 0)
    def _():
        out_ref[...] = wy

    @pl.when(e > 0)
    def _():
        out_ref[...] += wy


def kernel(x, gate_W, logit_bias, null_logit, W_gate, W_up, W_down, sg_W, su_W, sd_W):
    b, t, d = x.shape
    xf = x.reshape(_N, _D)

    combine, aux = pl.pallas_call(
        _router_kernel,
        out_shape=(
            jax.ShapeDtypeStruct((_N, _E), jnp.float32),
            jax.ShapeDtypeStruct((1, 1), jnp.float32),
        ),
    )(xf, gate_W.T, logit_bias.reshape(1, _E), null_logit.reshape(1, 1))

    ne = _E + 1
    xb = xf.astype(jnp.bfloat16)
    wg = jnp.concatenate([W_gate, sg_W.T[None]], axis=0).astype(jnp.bfloat16)
    wu = jnp.concatenate([W_up, su_W.T[None]], axis=0).astype(jnp.bfloat16)
    wd = jnp.concatenate([W_down, sd_W.T[None]], axis=0).astype(jnp.bfloat16)
    cmb = jnp.concatenate([combine, jnp.ones((_N, 1), jnp.float32)], axis=1).T.reshape(ne, 1, _N)

    nt = _N // _TT
    out = pl.pallas_call(
        _ffn_kernel,
        grid=(nt, ne),
        in_specs=[
            pl.BlockSpec((1, 1, _TT), lambda t, e: (e, 0, t)),
            pl.BlockSpec((_TT, _D), lambda t, e: (t, 0)),
            pl.BlockSpec((1, _D, _H), lambda t, e: (e, 0, 0)),
            pl.BlockSpec((1, _D, _H), lambda t, e: (e, 0, 0)),
            pl.BlockSpec((1, _H, _D), lambda t, e: (e, 0, 0)),
        ],
        out_specs=pl.BlockSpec((_TT, _D), lambda t, e: (t, 0)),
        out_shape=jax.ShapeDtypeStruct((_N, _D), jnp.float32),
        compiler_params=pltpu.CompilerParams(
            dimension_semantics=("arbitrary", "arbitrary"),
        ),
    )(cmb, xb, wg, wu, wd)

    return out.reshape(b, t, d), aux[0, 0]
